# Initial kernel scaffold; baseline (speedup 1.0000x reference)
import jax
import jax.numpy as jnp
from jax.experimental import pallas as pl


def kernel(rgb, depth, intrinsic, box, W1, b1, W2, b2):
    raise NotImplementedError("write your pallas kernel here")


# speedup vs baseline: 1.7603x; 1.7603x over previous
import jax
import jax.numpy as jnp
from jax.experimental import pallas as pl
from jax.experimental.pallas import tpu as pltpu

_B, _H, _W, _M = 4, 512, 512, 8
_HW = _H * _W
_LN = 4096
_SR = 8
_P = _SR * _LN
_NJ = _HW // _P


def _body(box_ref, w1t_ref, b1_ref, w2t_ref, b2_ref, x_ref, out_ref):
    bidx = pl.program_id(0)
    j = pl.program_id(1)
    w1t = w1t_ref[...]
    b1 = b1_ref[...]
    w2t = w2t_ref[...]
    b2 = b2_ref[...]
    preds = []
    for r in range(_SR):
        xtr = x_ref[:, 0, 0, r, :]
        ht = jax.lax.dot_general(
            w1t, xtr, (((1,), (0,)), ((), ())),
            preferred_element_type=jnp.float32)
        ht = jnp.maximum(ht + b1, 0.0)
        lt = jax.lax.dot_general(
            w2t, ht, (((1,), (0,)), ((), ())),
            preferred_element_type=jnp.float32)
        lt = lt + b2
        preds.append((lt[1:2, :] > lt[0:1, :]).astype(jnp.float32))
    pred1 = jnp.concatenate(preds, axis=0) > 0.5

    n = (j * _P
         + jax.lax.broadcasted_iota(jnp.int32, (_SR, _LN), 0) * _LN
         + jax.lax.broadcasted_iota(jnp.int32, (_SR, _LN), 1))
    v = n >> 9
    u = n & 511
    lab = jnp.ones((_SR, _LN), jnp.float32)
    for m in range(_M):
        x1 = box_ref[bidx, m, 0]
        y1 = box_ref[bidx, m, 1]
        x2 = box_ref[bidx, m, 2]
        y2 = box_ref[bidx, m, 3]
        lv = box_ref[bidx, m, 4].astype(jnp.float32)
        mask = (v >= x1) & (v <= x2) & (u >= y1) & (u <= y2) & pred1
        lab = jnp.where(mask, lv, lab)
    out_ref[0, 0] = lab


def kernel(rgb, depth, intrinsic, box, W1, b1, W2, b2):
    del depth, intrinsic
    xt5 = rgb.reshape(-1, 3).T.reshape(3, _B, _NJ, _SR, _LN)
    boxi = box.astype(jnp.int32)
    out = pl.pallas_call(
        _body,
        grid=(_B, _NJ),
        in_specs=[
            pl.BlockSpec(memory_space=pltpu.SMEM),
            pl.BlockSpec((64, 3), lambda b_, jj: (0, 0)),
            pl.BlockSpec((64, 1), lambda b_, jj: (0, 0)),
            pl.BlockSpec((2, 64), lambda b_, jj: (0, 0)),
            pl.BlockSpec((2, 1), lambda b_, jj: (0, 0)),
            pl.BlockSpec((3, 1, 1, _SR, _LN),
                         lambda b_, jj: (0, b_, jj, 0, 0)),
        ],
        out_specs=pl.BlockSpec((1, 1, _SR, _LN),
                               lambda b_, jj: (b_, jj, 0, 0)),
        out_shape=jax.ShapeDtypeStruct((_B, _NJ, _SR, _LN), jnp.float32),
    )(boxi, W1.T, b1.reshape(64, 1), W2.T, b2.reshape(2, 1), xt5)
    return out.reshape(_B, _H, _W)
